# baseline (speedup 1.0000x reference)
import functools

import jax
import jax.numpy as jnp
from jax import lax
from jax.experimental import pallas as pl
from jax.experimental.pallas import tpu as pltpu
from jax.experimental.pallas import tpu_sc as plsc

N = 1_000_000
M = 16
MAX_NORM = 1.0 - 1e-05
MN2 = MAX_NORM * MAX_NORM

NC, NS = 2, 16
NW = NC * NS
B_TOTAL = 16384 * 26
PER_W = B_TOTAL // NW
CH = 128
NG = PER_W // CH

_MAGIC = 0x5F3759DF


def _clip_chunk(rows_v, r, _):
    base = r * 16
    lanes = jnp.arange(16, dtype=jnp.int32)
    ridx = base + lanes
    cols = []
    acc = jnp.zeros((16,), jnp.float32)
    for j in range(M):
        cidx = jnp.bitwise_and(lanes + j, 15)
        v = plsc.load_gather(rows_v, [ridx, cidx])
        cols.append((cidx, v))
        acc = acc + v * v
    y = plsc.bitcast(_MAGIC - (plsc.bitcast(acc, jnp.int32) >> 1), jnp.float32)
    for _ in range(3):
        y = y * (1.5 - 0.5 * acc * y * y)
    scale = jnp.where(acc > MN2, MAX_NORM * y, 1.0)
    for cidx, v in cols:
        plsc.store_scatter(rows_v, [ridx, cidx], v * scale)
    return 0


def _body(x_hbm, w_hbm, out_hbm, idx_v, rows_v, gsem):
    wid = lax.axis_index("s") * NC + lax.axis_index("c")
    pltpu.sync_copy(x_hbm.at[wid], idx_v)

    def chunk(g, carry):
        pltpu.async_copy(w_hbm.at[idx_v.at[g]], rows_v, gsem).wait()
        lax.fori_loop(0, CH // 16, functools.partial(_clip_chunk, rows_v), 0)
        pltpu.sync_copy(rows_v, out_hbm.at[wid, g])
        return carry

    lax.fori_loop(0, NG, chunk, 0)


_mesh = plsc.VectorSubcoreMesh(
    core_axis_name="c", subcore_axis_name="s", num_cores=NC, num_subcores=NS
)

_embed = functools.partial(
    pl.kernel,
    out_type=jax.ShapeDtypeStruct((NW, NG, CH, M), jnp.float32),
    mesh=_mesh,
    scratch_types=[
        pltpu.VMEM((NG, CH), jnp.int32),
        pltpu.VMEM((CH, M), jnp.float32),
        pltpu.SemaphoreType.DMA,
    ],
    compiler_params=pltpu.CompilerParams(
        needs_layout_passes=False, use_tc_tiling_on_sc=False
    ),
)(_body)


def kernel(x, weight):
    xi = x.reshape(NW, NG, CH).astype(jnp.int32)
    out = _embed(xi, weight)
    return out.reshape(16384, 26, M)


# speedup vs baseline: 1.6643x; 1.6643x over previous
import functools

import jax
import jax.numpy as jnp
from jax import lax
from jax.experimental import pallas as pl
from jax.experimental.pallas import tpu as pltpu
from jax.experimental.pallas import tpu_sc as plsc

M = 16
F = 26
B = 16384
MAX_NORM = 1.0 - 1e-05
MN2 = MAX_NORM * MAX_NORM

NC, NS = 2, 16
NW = NC * NS
BW = B // NW
QG = BW // 128

_MAGIC = 0x5F3759DF


def _clip_group(rows_v, rows_t, r, _):
    base = r * 16
    lanes = jnp.arange(16, dtype=jnp.int32)
    ridx = base + lanes
    cols = []
    acc = jnp.zeros((16,), jnp.float32)
    for j in range(M):
        cidx = jnp.bitwise_and(lanes + j, 15)
        v = plsc.load_gather(rows_v, [ridx, cidx])
        cols.append((cidx, v))
        acc = acc + v * v
    y = plsc.bitcast(_MAGIC - (plsc.bitcast(acc, jnp.int32) >> 1), jnp.float32)
    for _ in range(3):
        y = y * (1.5 - 0.5 * acc * y * y)
    scale = jnp.where(acc > MN2, MAX_NORM * y, 1.0)
    for cidx, v in cols:
        plsc.store_scatter(rows_t, [cidx, ridx], v * scale)
    return 0


def _body(xt_hbm, w_hbm, out_hbm, idx_v, rows_v, rows_t, gsem):
    wid = lax.axis_index("s") * NC + lax.axis_index("c")
    b0 = wid * BW
    pltpu.sync_copy(xt_hbm.at[:, pl.ds(b0, BW)], idx_v)

    def per_f(f, carry):
        descs = [
            pltpu.async_copy(
                w_hbm.at[idx_v.at[f, pl.ds(q * 128, 128)]],
                rows_v.at[pl.ds(q * 128, 128)],
                gsem,
            )
            for q in range(QG)
        ]
        for d in descs:
            d.wait()
        lax.fori_loop(
            0, BW // 16, functools.partial(_clip_group, rows_v, rows_t), 0
        )
        pltpu.sync_copy(rows_t, out_hbm.at[f, :, pl.ds(b0, BW)])
        return carry

    lax.fori_loop(0, F, per_f, 0)


_mesh = plsc.VectorSubcoreMesh(
    core_axis_name="c", subcore_axis_name="s", num_cores=NC, num_subcores=NS
)

_embed = functools.partial(
    pl.kernel,
    out_type=jax.ShapeDtypeStruct((F, M, B), jnp.float32),
    mesh=_mesh,
    scratch_types=[
        pltpu.VMEM((F, BW), jnp.int32),
        pltpu.VMEM((BW, M), jnp.float32),
        pltpu.VMEM((M, BW), jnp.float32),
        pltpu.SemaphoreType.DMA,
    ],
    compiler_params=pltpu.CompilerParams(
        needs_layout_passes=False, use_tc_tiling_on_sc=False
    ),
)(_body)


def kernel(x, weight):
    xt = x.T.astype(jnp.int32)
    out3 = _embed(xt, weight)
    return out3.transpose(2, 0, 1)
